# Initial kernel scaffold; baseline (speedup 1.0000x reference)
import jax
import jax.numpy as jnp
from jax.experimental import pallas as pl


def kernel(x, edge_index, edge_weight, W, b, gamma, beta):
    raise NotImplementedError("write your pallas kernel here")


# speedup vs baseline: 2.7681x; 2.7681x over previous
import functools

import jax
import jax.numpy as jnp
from jax import lax
from jax.experimental import pallas as pl
from jax.experimental.pallas import tpu as pltpu
from jax.experimental.pallas import tpu_sc as plsc

N = 10000
C = 128
H = 64
E = 320000
NC = 2
NS = 16
G = 80
CHUNKS = E // NS // G
TILE_ROWS = 624
TAIL_ROWS = N - NS * TILE_ROWS
LANES = 16


def _bn_body(x_ref, gamma_ref, beta_ref, h_ref):
    xv = x_ref[...]
    mean = jnp.mean(xv, axis=0, keepdims=True)
    xc = xv - mean
    var = jnp.mean(xc * xc, axis=0, keepdims=True)
    hv = xc * lax.rsqrt(var + 1e-5) * gamma_ref[...] + beta_ref[...]
    h_ref[0] = hv[:, :H]
    h_ref[1] = hv[:, H:]


def _bn(x, gamma, beta):
    return pl.pallas_call(
        _bn_body,
        out_shape=jax.ShapeDtypeStruct((NC, N, H), jnp.float32),
    )(x, gamma, beta)


def _spmm_body(h_ref, src_ref, dst_ref, w_ref, zero_ref, out_ref,
               src_v, dst_v, w_v, rows_v, acc_sh, sem):
    cid = lax.axis_index("c")
    sid = lax.axis_index("s")
    pltpu.sync_copy(src_ref.at[sid], src_v)
    pltpu.sync_copy(dst_ref.at[sid], dst_v)
    pltpu.sync_copy(w_ref.at[sid], w_v)
    pltpu.sync_copy(zero_ref, acc_sh.at[pl.ds(sid * TILE_ROWS, TILE_ROWS)])

    @pl.when(sid == NS - 1)
    def _():
        pltpu.sync_copy(zero_ref.at[pl.ds(0, TAIL_ROWS)],
                        acc_sh.at[pl.ds(NS * TILE_ROWS, TAIL_ROWS)])

    plsc.subcore_barrier()

    for cc in range(NC):
        @pl.when(cid == cc)
        def _(cc=cc):
            h_hbm = h_ref.at[cc]

            def chunk(j, carry):
                pltpu.async_copy(h_hbm.at[src_v.at[j]], rows_v, sem).wait()

                def group(g, carry2):
                    wrow = w_v[j, pl.ds(g * LANES, LANES)]
                    for l in range(LANES):
                        e = g * LANES + l
                        for c4 in range(H // LANES):
                            sl = pl.ds(c4 * LANES, LANES)
                            rows_v[e, sl] = rows_v[e, sl] * wrow[l]
                    return carry2

                lax.fori_loop(0, G // LANES, group, 0)
                pltpu.sync_copy(rows_v, acc_sh.at[dst_v.at[j]], add=True)
                return carry

            lax.fori_loop(0, CHUNKS, chunk, 0)

    plsc.subcore_barrier()
    for cc in range(NC):
        @pl.when(cid == cc)
        def _(cc=cc):
            pltpu.sync_copy(acc_sh.at[pl.ds(sid * TILE_ROWS, TILE_ROWS)],
                            out_ref.at[cc].at[pl.ds(sid * TILE_ROWS, TILE_ROWS)])

            @pl.when(sid == NS - 1)
            def _():
                pltpu.sync_copy(acc_sh.at[pl.ds(NS * TILE_ROWS, TAIL_ROWS)],
                                out_ref.at[cc].at[pl.ds(NS * TILE_ROWS, TAIL_ROWS)])


@functools.cache
def _make_spmm():
    return pl.kernel(
        _spmm_body,
        out_type=jax.ShapeDtypeStruct((NC, N, H), jnp.float32),
        mesh=plsc.VectorSubcoreMesh(core_axis_name="c", subcore_axis_name="s",
                                    num_cores=NC, num_subcores=NS),
        compiler_params=pltpu.CompilerParams(use_tc_tiling_on_sc=False),
        scratch_types=[
            pltpu.VMEM((CHUNKS, G), jnp.int32),
            pltpu.VMEM((CHUNKS, G), jnp.int32),
            pltpu.VMEM((CHUNKS, G), jnp.float32),
            pltpu.VMEM((G, H), jnp.float32),
            pltpu.VMEM_SHARED((N, H), jnp.float32),
            pltpu.SemaphoreType.DMA,
        ],
    )


def _mm_body(h_ref, t1_ref, u_ref, wa_ref, w1_ref, wc_ref, b_ref, o_ref):
    hv = jnp.concatenate([h_ref[0], h_ref[1]], axis=1)
    t1v = jnp.concatenate([t1_ref[0], t1_ref[1]], axis=1)
    uv = jnp.concatenate([u_ref[0], u_ref[1]], axis=1)
    acc = jnp.dot(hv, wa_ref[...], preferred_element_type=jnp.float32)
    acc = acc + jnp.dot(t1v, w1_ref[...], preferred_element_type=jnp.float32)
    acc = acc + jnp.dot(uv, wc_ref[...], preferred_element_type=jnp.float32)
    acc = acc + b_ref[...]
    o_ref[...] = jnp.maximum(acc, 0.0)


def _mm(h, t1, u, wa, w1, wc, b):
    return pl.pallas_call(
        _mm_body,
        out_shape=jax.ShapeDtypeStruct((N, C), jnp.float32),
    )(h, t1, u, wa, w1, wc, b)


def kernel(x, edge_index, edge_weight, W, b, gamma, beta):
    src = edge_index[0].reshape(NS, CHUNKS, G)
    dst = edge_index[1].reshape(NS, CHUNKS, G)
    w3 = edge_weight.reshape(NS, CHUNKS, G)
    zeros = jnp.zeros((TILE_ROWS, H), jnp.float32)
    h = _bn(x, gamma.reshape(1, C), beta.reshape(1, C))
    spmm = _make_spmm()
    t1 = spmm(h, src, dst, w3, zeros)
    u = spmm(t1, src, dst, w3, zeros)
    wa = W[0] - W[2]
    wc = 2.0 * W[2]
    return _mm(h, t1, u, wa, W[1], wc, b.reshape(1, C))


# speedup vs baseline: 6.9321x; 2.5043x over previous
import functools

import jax
import jax.numpy as jnp
from jax import lax
from jax.experimental import pallas as pl
from jax.experimental.pallas import tpu as pltpu
from jax.experimental.pallas import tpu_sc as plsc

N = 10000
C = 128
H = 64
E = 320000
NC = 2
NS = 16
G = 80
CHUNKS = E // NS // G
TILE_ROWS = 624
TAIL_ROWS = N - NS * TILE_ROWS
LANES = 16


def _bn_body(x_ref, gamma_ref, beta_ref, h_ref):
    xv = x_ref[...]
    mean = jnp.mean(xv, axis=0, keepdims=True)
    xc = xv - mean
    var = jnp.mean(xc * xc, axis=0, keepdims=True)
    hv = xc * lax.rsqrt(var + 1e-5) * gamma_ref[...] + beta_ref[...]
    h_ref[0] = hv[:, :H]
    h_ref[1] = hv[:, H:]


def _bn(x, gamma, beta):
    return pl.pallas_call(
        _bn_body,
        out_shape=jax.ShapeDtypeStruct((NC, N, H), jnp.float32),
    )(x, gamma, beta)


def _scale_chunk(w_v, rows_v, j):
    for g in range(G // LANES):
        wrow = w_v[j, pl.ds(g * LANES, LANES)]
        for l in range(LANES):
            e = g * LANES + l
            for c4 in range(H // LANES):
                sl = pl.ds(c4 * LANES, LANES)
                rows_v[e, sl] = rows_v[e, sl] * wrow[l]


def _spmm_body(h_ref, src_ref, dst_ref, w_ref, zero_ref, out_ref,
               src_v, dst_v, w_v, rows_v, rows_v2, acc_sh,
               gsem0, gsem1, ssem0, ssem1):
    cid = lax.axis_index("c")
    sid = lax.axis_index("s")
    pltpu.sync_copy(src_ref.at[sid], src_v)
    pltpu.sync_copy(dst_ref.at[sid], dst_v)
    pltpu.sync_copy(w_ref.at[sid], w_v)
    pltpu.sync_copy(zero_ref, acc_sh.at[pl.ds(sid * TILE_ROWS, TILE_ROWS)])

    @pl.when(sid == NS - 1)
    def _():
        pltpu.sync_copy(zero_ref.at[pl.ds(0, TAIL_ROWS)],
                        acc_sh.at[pl.ds(NS * TILE_ROWS, TAIL_ROWS)])

    plsc.subcore_barrier()

    for cc in range(NC):
        @pl.when(cid == cc)
        def _(cc=cc):
            h_hbm = h_ref.at[cc]

            def chunk2(jj, carry):
                j0 = 2 * jj
                j1 = 2 * jj + 1
                g0 = pltpu.async_copy(h_hbm.at[src_v.at[j0]], rows_v, gsem0)
                g1 = pltpu.async_copy(h_hbm.at[src_v.at[j1]], rows_v2, gsem1)
                g0.wait()
                _scale_chunk(w_v, rows_v, j0)
                s0 = pltpu.async_copy(rows_v, acc_sh.at[dst_v.at[j0]],
                                      ssem0, add=True)
                g1.wait()
                _scale_chunk(w_v, rows_v2, j1)
                s1 = pltpu.async_copy(rows_v2, acc_sh.at[dst_v.at[j1]],
                                      ssem1, add=True)
                s0.wait()
                s1.wait()
                return carry

            lax.fori_loop(0, CHUNKS // 2, chunk2, 0)

    plsc.subcore_barrier()
    for cc in range(NC):
        @pl.when(cid == cc)
        def _(cc=cc):
            pltpu.sync_copy(acc_sh.at[pl.ds(sid * TILE_ROWS, TILE_ROWS)],
                            out_ref.at[cc].at[pl.ds(sid * TILE_ROWS, TILE_ROWS)])

            @pl.when(sid == NS - 1)
            def _():
                pltpu.sync_copy(acc_sh.at[pl.ds(NS * TILE_ROWS, TAIL_ROWS)],
                                out_ref.at[cc].at[pl.ds(NS * TILE_ROWS, TAIL_ROWS)])


@functools.cache
def _make_spmm():
    return pl.kernel(
        _spmm_body,
        out_type=jax.ShapeDtypeStruct((NC, N, H), jnp.float32),
        mesh=plsc.VectorSubcoreMesh(core_axis_name="c", subcore_axis_name="s",
                                    num_cores=NC, num_subcores=NS),
        compiler_params=pltpu.CompilerParams(use_tc_tiling_on_sc=False),
        scratch_types=[
            pltpu.VMEM((CHUNKS, G), jnp.int32),
            pltpu.VMEM((CHUNKS, G), jnp.int32),
            pltpu.VMEM((CHUNKS, G), jnp.float32),
            pltpu.VMEM((G, H), jnp.float32),
            pltpu.VMEM((G, H), jnp.float32),
            pltpu.VMEM_SHARED((N, H), jnp.float32),
            pltpu.SemaphoreType.DMA,
            pltpu.SemaphoreType.DMA,
            pltpu.SemaphoreType.DMA,
            pltpu.SemaphoreType.DMA,
        ],
    )


def _mm_body(h_ref, t1_ref, u_ref, wa_ref, w1_ref, wc_ref, b_ref, o_ref):
    hv = jnp.concatenate([h_ref[0], h_ref[1]], axis=1)
    t1v = jnp.concatenate([t1_ref[0], t1_ref[1]], axis=1)
    uv = jnp.concatenate([u_ref[0], u_ref[1]], axis=1)
    acc = jnp.dot(hv, wa_ref[...], preferred_element_type=jnp.float32)
    acc = acc + jnp.dot(t1v, w1_ref[...], preferred_element_type=jnp.float32)
    acc = acc + jnp.dot(uv, wc_ref[...], preferred_element_type=jnp.float32)
    acc = acc + b_ref[...]
    o_ref[...] = jnp.maximum(acc, 0.0)


def _mm(h, t1, u, wa, w1, wc, b):
    return pl.pallas_call(
        _mm_body,
        out_shape=jax.ShapeDtypeStruct((N, C), jnp.float32),
    )(h, t1, u, wa, w1, wc, b)


def kernel(x, edge_index, edge_weight, W, b, gamma, beta):
    src = edge_index[0].reshape(NS, CHUNKS, G)
    dst = edge_index[1].reshape(NS, CHUNKS, G)
    w3 = edge_weight.reshape(NS, CHUNKS, G)
    zeros = jnp.zeros((TILE_ROWS, H), jnp.float32)
    h = _bn(x, gamma.reshape(1, C), beta.reshape(1, C))
    spmm = _make_spmm()
    t1 = spmm(h, src, dst, w3, zeros)
    u = spmm(t1, src, dst, w3, zeros)
    wa = W[0] - W[2]
    wc = 2.0 * W[2]
    return _mm(h, t1, u, wa, W[1], wc, b.reshape(1, C))


# speedup vs baseline: 9.0728x; 1.3088x over previous
import functools

import jax
import jax.numpy as jnp
from jax import lax
from jax.experimental import pallas as pl
from jax.experimental.pallas import tpu as pltpu
from jax.experimental.pallas import tpu_sc as plsc

N = 10000
C = 128
H = 64
E = 320000
NC = 2
NS = 16
G = 80
CHUNKS = E // NS // G
TILE_ROWS = 624
TAIL_ROWS = N - NS * TILE_ROWS
LANES = 16


def _bn_body(x_ref, gamma_ref, beta_ref, h_ref):
    xv = x_ref[...]
    mean = jnp.mean(xv, axis=0, keepdims=True)
    xc = xv - mean
    var = jnp.mean(xc * xc, axis=0, keepdims=True)
    hv = xc * lax.rsqrt(var + 1e-5) * gamma_ref[...] + beta_ref[...]
    h_ref[0] = hv[:, :H]
    h_ref[1] = hv[:, H:]


def _bn(x, gamma, beta):
    return pl.pallas_call(
        _bn_body,
        out_shape=jax.ShapeDtypeStruct((NC, N, H), jnp.float32),
    )(x, gamma, beta)


def _scale_chunk(w_v, rows_v, j):
    for g in range(G // LANES):
        wrow = w_v[j, pl.ds(g * LANES, LANES)]
        for l in range(LANES):
            e = g * LANES + l
            for c4 in range(H // LANES):
                sl = pl.ds(c4 * LANES, LANES)
                rows_v[e, sl] = rows_v[e, sl] * wrow[l]


NBUF = 5


def _spmm_body(h_ref, src_ref, dst_ref, w_ref, zero_ref, out_ref,
               src_v, dst_v, w_v, bufs, acc_sh, gsems, ssems):
    cid = lax.axis_index("c")
    sid = lax.axis_index("s")
    pltpu.sync_copy(src_ref.at[sid], src_v)
    pltpu.sync_copy(dst_ref.at[sid], dst_v)
    pltpu.sync_copy(w_ref.at[sid], w_v)
    pltpu.sync_copy(zero_ref, acc_sh.at[pl.ds(sid * TILE_ROWS, TILE_ROWS)])

    @pl.when(sid == NS - 1)
    def _():
        pltpu.sync_copy(zero_ref.at[pl.ds(0, TAIL_ROWS)],
                        acc_sh.at[pl.ds(NS * TILE_ROWS, TAIL_ROWS)])

    plsc.subcore_barrier()

    for cc in range(NC):
        @pl.when(cid == cc)
        def _(cc=cc):
            h_hbm = h_ref.at[cc]

            def gather(b, j):
                return pltpu.make_async_copy(
                    h_hbm.at[src_v.at[j]], bufs.at[b], gsems.at[b])

            def scatter(b, j):
                return pltpu.make_async_copy(
                    bufs.at[b], acc_sh.at[dst_v.at[j]], ssems.at[b])

            for b in range(NBUF):
                gather(b, b).start()

            def round_(r, carry):
                for b in range(NBUF):
                    j = NBUF * r + b
                    gather(b, j).wait()
                    _scale_chunk(w_v, bufs.at[b], j)
                    scatter(b, j).start(add=True)
                    bp = (b - 1) % NBUF
                    jp = j - 1

                    @pl.when((jp >= 0) & (jp + NBUF < CHUNKS))
                    def _():
                        scatter(bp, jp).wait()
                        gather(bp, jp + NBUF).start()
                return carry

            lax.fori_loop(0, CHUNKS // NBUF, round_, 0)
            for b in range(NBUF):
                scatter(b, CHUNKS - NBUF + b).wait()

    plsc.subcore_barrier()
    for cc in range(NC):
        @pl.when(cid == cc)
        def _(cc=cc):
            pltpu.sync_copy(acc_sh.at[pl.ds(sid * TILE_ROWS, TILE_ROWS)],
                            out_ref.at[cc].at[pl.ds(sid * TILE_ROWS, TILE_ROWS)])

            @pl.when(sid == NS - 1)
            def _():
                pltpu.sync_copy(acc_sh.at[pl.ds(NS * TILE_ROWS, TAIL_ROWS)],
                                out_ref.at[cc].at[pl.ds(NS * TILE_ROWS, TAIL_ROWS)])


@functools.cache
def _make_spmm():
    return pl.kernel(
        _spmm_body,
        out_type=jax.ShapeDtypeStruct((NC, N, H), jnp.float32),
        mesh=plsc.VectorSubcoreMesh(core_axis_name="c", subcore_axis_name="s",
                                    num_cores=NC, num_subcores=NS),
        compiler_params=pltpu.CompilerParams(use_tc_tiling_on_sc=False),
        scratch_types=[
            pltpu.VMEM((CHUNKS, G), jnp.int32),
            pltpu.VMEM((CHUNKS, G), jnp.int32),
            pltpu.VMEM((CHUNKS, G), jnp.float32),
            pltpu.VMEM((NBUF, G, H), jnp.float32),
            pltpu.VMEM_SHARED((N, H), jnp.float32),
            pltpu.SemaphoreType.DMA((NBUF,)),
            pltpu.SemaphoreType.DMA((NBUF,)),
        ],
    )


def _mm_body(h_ref, t1_ref, u_ref, wa_ref, w1_ref, wc_ref, b_ref, o_ref):
    hv = jnp.concatenate([h_ref[0], h_ref[1]], axis=1)
    t1v = jnp.concatenate([t1_ref[0], t1_ref[1]], axis=1)
    uv = jnp.concatenate([u_ref[0], u_ref[1]], axis=1)
    acc = jnp.dot(hv, wa_ref[...], preferred_element_type=jnp.float32)
    acc = acc + jnp.dot(t1v, w1_ref[...], preferred_element_type=jnp.float32)
    acc = acc + jnp.dot(uv, wc_ref[...], preferred_element_type=jnp.float32)
    acc = acc + b_ref[...]
    o_ref[...] = jnp.maximum(acc, 0.0)


def _mm(h, t1, u, wa, w1, wc, b):
    return pl.pallas_call(
        _mm_body,
        out_shape=jax.ShapeDtypeStruct((N, C), jnp.float32),
    )(h, t1, u, wa, w1, wc, b)


def kernel(x, edge_index, edge_weight, W, b, gamma, beta):
    src = edge_index[0].reshape(NS, CHUNKS, G)
    dst = edge_index[1].reshape(NS, CHUNKS, G)
    w3 = edge_weight.reshape(NS, CHUNKS, G)
    zeros = jnp.zeros((TILE_ROWS, H), jnp.float32)
    h = _bn(x, gamma.reshape(1, C), beta.reshape(1, C))
    spmm = _make_spmm()
    t1 = spmm(h, src, dst, w3, zeros)
    u = spmm(t1, src, dst, w3, zeros)
    wa = W[0] - W[2]
    wc = 2.0 * W[2]
    return _mm(h, t1, u, wa, W[1], wc, b.reshape(1, C))
